# Initial kernel scaffold; baseline (speedup 1.0000x reference)
import functools

import jax
import jax.numpy as jnp
from jax import lax
from jax.experimental import pallas as pl
from jax.experimental.pallas import tpu as pltpu
from jax.experimental.pallas import tpu_sc as plsc

D = 64
B = 4096
S = 200
NT = 10

NC = 2
NS = 16
NW = NC * NS
B_PER_W = B // NW


@functools.partial(
    pl.kernel,
    out_type=jax.ShapeDtypeStruct((B, S, D), jnp.float32),
    mesh=plsc.VectorSubcoreMesh(core_axis_name="c", subcore_axis_name="s"),
    scratch_types=[
        pltpu.VMEM((192,), jnp.int32),
        pltpu.VMEM((S, D), jnp.float32),
        pltpu.VMEM((16, D), jnp.float32),
        pltpu.SemaphoreType.DMA,
    ],
)
def _soft_embed(tokens_hbm, wte_hbm, learned_hbm, out_hbm,
                idx_v, row_v, learned_v, sem):
    wid = lax.axis_index("s") * NC + lax.axis_index("c")
    base = wid * B_PER_W

    pltpu.sync_copy(learned_hbm, learned_v.at[pl.ds(0, NT)])
    pltpu.sync_copy(learned_hbm.at[pl.ds(0, 8)], row_v.at[pl.ds(0, 8)])

    def body(i, carry):
        b = base + i
        pltpu.sync_copy(tokens_hbm.at[b, pl.ds(8, 192)], idx_v)
        pltpu.async_copy(wte_hbm.at[idx_v.at[pl.ds(0, 128)]],
                         row_v.at[pl.ds(8, 128)], sem).wait()
        pltpu.async_copy(wte_hbm.at[idx_v.at[pl.ds(128, 64)]],
                         row_v.at[pl.ds(136, 64)], sem).wait()
        for r in range(8, NT):
            for c in range(0, D, 16):
                row_v[r, pl.ds(c, 16)] = learned_v[r, pl.ds(c, 16)]
        pltpu.sync_copy(row_v, out_hbm.at[b])
        return carry

    lax.fori_loop(0, B_PER_W, body, 0)


def kernel(tokens, wte_weight, learned_embedding):
    return _soft_embed(tokens, wte_weight, learned_embedding)


# speedup vs baseline: 1.2625x; 1.2625x over previous
import functools

import jax
import jax.numpy as jnp
from jax import lax
from jax.experimental import pallas as pl
from jax.experimental.pallas import tpu as pltpu
from jax.experimental.pallas import tpu_sc as plsc

D = 64
B = 4096
S = 200
NT = 10

NC = 2
NS = 16
NW = NC * NS
B_PER_W = B // NW


@functools.partial(
    pl.kernel,
    out_type=jax.ShapeDtypeStruct((B * S, D), jnp.float32),
    mesh=plsc.VectorSubcoreMesh(core_axis_name="c", subcore_axis_name="s"),
    compiler_params=pltpu.CompilerParams(use_tc_tiling_on_sc=False),
    scratch_types=[
        pltpu.VMEM((192,), jnp.int32),
        pltpu.VMEM((S, D), jnp.float32),
        pltpu.VMEM((16, D), jnp.float32),
        pltpu.SemaphoreType.DMA,
    ],
)
def _soft_embed(tokens_hbm, wte_hbm, learned_hbm, out_hbm,
                idx_v, row_v, learned_v, sem):
    wid = lax.axis_index("s") * NC + lax.axis_index("c")
    base = wid * B_PER_W

    pltpu.sync_copy(learned_hbm, learned_v.at[pl.ds(0, NT)])
    pltpu.sync_copy(learned_hbm.at[pl.ds(0, 8)], row_v.at[pl.ds(0, 8)])

    def body(i, carry):
        b = base + i
        pltpu.sync_copy(tokens_hbm.at[pl.ds(b * S + 8, 192)], idx_v)
        pltpu.async_copy(wte_hbm.at[idx_v.at[pl.ds(0, 128)]],
                         row_v.at[pl.ds(8, 128)], sem).wait()
        pltpu.async_copy(wte_hbm.at[idx_v.at[pl.ds(128, 64)]],
                         row_v.at[pl.ds(136, 64)], sem).wait()
        for r in range(8, NT):
            for c in range(0, D, 16):
                row_v[r, pl.ds(c, 16)] = learned_v[r, pl.ds(c, 16)]
        pltpu.sync_copy(row_v, out_hbm.at[pl.ds(b * S, S)])
        return carry

    lax.fori_loop(0, B_PER_W, body, 0)


def kernel(tokens, wte_weight, learned_embedding):
    out = _soft_embed(tokens.reshape(B * S), wte_weight, learned_embedding)
    return out.reshape(B, S, D)


# speedup vs baseline: 1.4971x; 1.1858x over previous
import functools

import jax
import jax.numpy as jnp
from jax import lax
from jax.experimental import pallas as pl
from jax.experimental.pallas import tpu as pltpu
from jax.experimental.pallas import tpu_sc as plsc

D = 64
B = 4096
S = 200
NT = 10

NC = 2
NS = 16
NW = NC * NS
B_PER_W = B // NW
NB = 2
N_IT = B_PER_W // NB
CHUNK = NB * S


@functools.partial(
    pl.kernel,
    out_type=jax.ShapeDtypeStruct((B * S, D), jnp.float32),
    mesh=plsc.VectorSubcoreMesh(core_axis_name="c", subcore_axis_name="s"),
    compiler_params=pltpu.CompilerParams(use_tc_tiling_on_sc=False),
    scratch_types=[
        pltpu.VMEM((CHUNK,), jnp.int32),
        pltpu.VMEM((CHUNK,), jnp.int32),
        pltpu.VMEM((CHUNK, D), jnp.float32),
        pltpu.VMEM((CHUNK, D), jnp.float32),
        pltpu.VMEM((16, D), jnp.float32),
        pltpu.SemaphoreType.DMA,
        pltpu.SemaphoreType.DMA,
        pltpu.SemaphoreType.DMA,
        pltpu.SemaphoreType.DMA,
        pltpu.SemaphoreType.DMA,
        pltpu.SemaphoreType.DMA,
    ],
)
def _soft_embed(tokens_hbm, wte_hbm, learned_hbm, out_hbm,
                idx0, idx1, row0, row1, learned_v,
                si0, si1, sg0, sg1, so0, so1):
    idx = (idx0, idx1)
    row = (row0, row1)
    sem_i = (si0, si1)
    sem_g = (sg0, sg1)
    sem_o = (so0, so1)

    wid = lax.axis_index("s") * NC + lax.axis_index("c")
    base = wid * B_PER_W

    pltpu.sync_copy(learned_hbm, learned_v.at[pl.ds(0, NT)])
    for s in range(2):
        for j in range(NB):
            pltpu.sync_copy(learned_hbm.at[pl.ds(0, 8)],
                            row[s].at[pl.ds(j * S, 8)])

    def start_idx(it, s):
        cb = base + it * NB
        pltpu.async_copy(tokens_hbm.at[pl.ds(cb * S, CHUNK)], idx[s], sem_i[s])

    for s in range(2):
        start_idx(s, s)

    def body(k, carry):
        i0 = k * 2
        for s in range(2):
            it = i0 + s
            cb = base + it * NB
            pltpu.make_async_copy(tokens_hbm.at[pl.ds(0, CHUNK)],
                                  idx[s], sem_i[s]).wait()
            @pl.when(it >= 2)
            def _():
                pltpu.make_async_copy(row[s], out_hbm.at[pl.ds(0, CHUNK)],
                                      sem_o[s]).wait()
            for j in range(NB):
                pltpu.async_copy(wte_hbm.at[idx[s].at[pl.ds(j * S + 8, 128)]],
                                 row[s].at[pl.ds(j * S + 8, 128)], sem_g[s])
                pltpu.async_copy(wte_hbm.at[idx[s].at[pl.ds(j * S + 136, 64)]],
                                 row[s].at[pl.ds(j * S + 136, 64)], sem_g[s])
            for j in range(NB):
                pltpu.make_async_copy(wte_hbm.at[pl.ds(0, 192)],
                                      row[s].at[pl.ds(j * S + 8, 192)],
                                      sem_g[s]).wait()
            @pl.when(it + 2 < N_IT)
            def _():
                start_idx(it + 2, s)
            for j in range(NB):
                for r in range(8, NT):
                    for c in range(0, D, 16):
                        row[s][j * S + r, pl.ds(c, 16)] = \
                            learned_v[r, pl.ds(c, 16)]
            pltpu.async_copy(row[s], out_hbm.at[pl.ds(cb * S, CHUNK)],
                             sem_o[s])
        return carry

    lax.fori_loop(0, N_IT // 2, body, 0)

    for s in range(2):
        pltpu.make_async_copy(row[s], out_hbm.at[pl.ds(0, CHUNK)],
                              sem_o[s]).wait()


def kernel(tokens, wte_weight, learned_embedding):
    out = _soft_embed(tokens.reshape(B * S), wte_weight, learned_embedding)
    return out.reshape(B, S, D)
